# baseline (speedup 1.0000x reference)
import jax
import jax.numpy as jnp
from jax import lax
from jax.experimental import pallas as pl
from jax.experimental.pallas import tpu as pltpu
from jax.experimental.pallas import tpu_sc as plsc

EMB_DIM = 32
BATCH_N = 16384
_NC, _NS = 2, 16
_NW = _NC * _NS
_BPW = BATCH_N // _NW
_BB = 2048
_JB = 32768
_SB = _JB // 4


def _cdiv(a, b):
    return (a + b - 1) // b


def _repack_body(x_ref, ox_ref):
    x = x_ref[...]
    x4 = jnp.concatenate(
        [x[:, a * _SB:(a + 1) * _SB] for a in range(4)],
        axis=0)
    ox_ref[...] = jnp.transpose(x4)


def _repack1(mt):
    n = mt.shape[1]
    grid = _cdiv(n, _JB)
    spec_in = pl.BlockSpec((EMB_DIM, _JB), lambda j: (0, j))
    spec_out = pl.BlockSpec((_SB, 128), lambda j: (j, 0))
    shape = jax.ShapeDtypeStruct((grid * _SB, 128), jnp.float32)
    return pl.pallas_call(
        _repack_body,
        grid=(grid,),
        in_specs=[spec_in],
        out_specs=spec_out,
        out_shape=shape,
    )(mt)


def _gather_body(n4_hbm, pa, ga, idx, buf, sem):
    wid = lax.axis_index("s") * _NC + lax.axis_index("c")
    base = wid * _BPW
    pltpu.sync_copy(n4_hbm.at[pl.ds(base, _BPW)], idx)
    pltpu.async_copy(pa.at[idx], buf, sem).wait()
    pltpu.sync_copy(buf, ga.at[pl.ds(base, _BPW)])


def _sc_gather1(n4, pa):
    row = jax.ShapeDtypeStruct((BATCH_N, 128), jnp.float32)
    mesh = plsc.VectorSubcoreMesh(core_axis_name="c", subcore_axis_name="s")
    return pl.kernel(
        _gather_body,
        out_type=row,
        mesh=mesh,
        compiler_params=pltpu.CompilerParams(use_tc_tiling_on_sc=True),
        scratch_types=[
            pltpu.VMEM((_BPW,), jnp.int32),
            pltpu.VMEM((_BPW, 128), jnp.float32),
            pltpu.SemaphoreType.DMA,
        ],
    )(n4, pa)


def _mlp_body(gug, gig, gum, gim, ku, ki,
              W0u4, W0i4, b0, W1, b1, W2, b2, W3, b3, Wpg4, Wph, bp, out):
    lane_blk = lax.broadcasted_iota(jnp.int32, (1, 128), 1) // EMB_DIM
    mu_v = (ku[...] == lane_blk).astype(jnp.float32)
    mi_v = (ki[...] == lane_blk).astype(jnp.float32)
    mm = lambda a, b: lax.dot_general(
        a, b, (((1,), (0,)), ((), ())), preferred_element_type=jnp.float32)
    h = jnp.maximum(
        mm(gum[...] * mu_v, W0u4[...]) + mm(gim[...] * mi_v, W0i4[...])
        + b0[...], 0.0)
    for W, b in ((W1, b1), (W2, b2), (W3, b3)):
        h = jnp.maximum(
            lax.dot_general(h, W[...], (((1,), (1,)), ((), ())),
                            preferred_element_type=jnp.float32) + b[...],
            0.0)
    mig = gig[...] * mi_v
    ig_rep = mm(mig, jnp.tile(jnp.eye(EMB_DIM, dtype=jnp.float32), (4, 4)))
    s = (gug[...] * mu_v) * ig_rep
    logits = (mm(s, Wpg4[...]) +
              lax.dot_general(h, Wph[...], (((1,), (1,)), ((), ())),
                              preferred_element_type=jnp.float32)
              + bp[...].reshape(1, 1))
    out[...] = jax.nn.sigmoid(logits)


def _tc_mlp(gug, gig, gum, gim, ku, ki,
            W0u4, W0i4, b0, W1, b1, W2, b2, W3, b3, Wpg4, Wph, bp,
            interpret=False):
    act = pl.BlockSpec((_BB, 128), lambda b: (b, 0))
    col = pl.BlockSpec((_BB, 1), lambda b: (b, 0))
    full2 = lambda a: pl.BlockSpec(a.shape, lambda b: (0,) * a.ndim)
    grid = BATCH_N // _BB
    return pl.pallas_call(
        _mlp_body,
        grid=(grid,),
        in_specs=[act, act, act, act, col, col,
                  full2(W0u4), full2(W0i4), full2(b0), full2(W1), full2(b1),
                  full2(W2), full2(b2), full2(W3), full2(b3),
                  full2(Wpg4), full2(Wph), full2(bp)],
        out_specs=pl.BlockSpec((_BB, 1), lambda b: (b, 0)),
        out_shape=jax.ShapeDtypeStruct((BATCH_N, 1), jnp.float32),
        interpret=interpret,
    )(gug, gig, gum, gim, ku, ki,
      W0u4, W0i4, b0, W1, b1, W2, b2, W3, b3, Wpg4, Wph, bp)


def kernel(u, i, user_gmf, item_gmf, user_mlp, item_mlp,
           W0, b0, W1, b1, W2, b2, W3, b3, Wp, bp):
    u = u.astype(jnp.int32)
    i = i.astype(jnp.int32)
    u4 = (u // _JB) * _SB + (u % _SB)
    i4 = (i // _JB) * _SB + (i % _SB)
    ku = ((u % _JB) // _SB).reshape(BATCH_N, 1)
    ki = ((i % _JB) // _SB).reshape(BATCH_N, 1)
    W0u4 = jnp.tile(W0[:, :EMB_DIM].T, (4, 1))
    W0i4 = jnp.tile(W0[:, EMB_DIM:].T, (4, 1))
    Wpg4 = jnp.tile(Wp[:, :EMB_DIM].T, (4, 1))
    Wph = Wp[:, EMB_DIM:]
    pug = _repack1(user_gmf.T)
    gug = _sc_gather1(u4, pug)
    pum = _repack1(user_mlp.T)
    gum = _sc_gather1(u4, pum)
    pig = _repack1(item_gmf.T)
    gig = _sc_gather1(i4, pig)
    pim = _repack1(item_mlp.T)
    gim = _sc_gather1(i4, pim)
    out = _tc_mlp(gug, gig, gum, gim, ku, ki,
                  W0u4, W0i4, b0, W1, b1, W2, b2, W3, b3, Wpg4, Wph, bp)
    return jnp.squeeze(out, axis=-1)


# speedup vs baseline: 1.0375x; 1.0375x over previous
import jax
import jax.numpy as jnp
from jax import lax
from jax.experimental import pallas as pl
from jax.experimental.pallas import tpu as pltpu
from jax.experimental.pallas import tpu_sc as plsc

EMB_DIM = 32
BATCH_N = 16384
_NC, _NS = 2, 16
_NW = _NC * _NS
_BPW = BATCH_N // _NW
_BB = 4096
_JB = 32768
_SB = _JB // 4


def _cdiv(a, b):
    return (a + b - 1) // b


def _repack_body(x_ref, y_ref, ox_ref, oy_ref):
    for ref, oref in ((x_ref, ox_ref), (y_ref, oy_ref)):
        x = ref[...]
        x4 = jnp.concatenate(
            [x[:, a * _SB:(a + 1) * _SB] for a in range(4)],
            axis=0)
        oref[...] = jnp.transpose(x4)


def _repack2(mt_a, mt_b):
    n = mt_a.shape[1]
    grid = _cdiv(n, _JB)
    spec_in = pl.BlockSpec((EMB_DIM, _JB), lambda j: (0, j))
    spec_out = pl.BlockSpec((_SB, 128), lambda j: (j, 0))
    shape = jax.ShapeDtypeStruct((grid * _SB, 128), jnp.float32)
    return pl.pallas_call(
        _repack_body,
        grid=(grid,),
        in_specs=[spec_in, spec_in],
        out_specs=[spec_out, spec_out],
        out_shape=[shape, shape],
    )(mt_a, mt_b)


def _gather_body(n4_hbm, pa, pb, ga, gb, idx, buf, sem):
    wid = lax.axis_index("s") * _NC + lax.axis_index("c")
    base = wid * _BPW
    pltpu.sync_copy(n4_hbm.at[pl.ds(base, _BPW)], idx)
    for tbl, out in ((pa, ga), (pb, gb)):
        pltpu.async_copy(tbl.at[idx], buf, sem).wait()
        pltpu.sync_copy(buf, out.at[pl.ds(base, _BPW)])


def _sc_gather2(n4, pa, pb):
    row = jax.ShapeDtypeStruct((BATCH_N, 128), jnp.float32)
    mesh = plsc.VectorSubcoreMesh(core_axis_name="c", subcore_axis_name="s")
    return pl.kernel(
        _gather_body,
        out_type=(row, row),
        mesh=mesh,
        compiler_params=pltpu.CompilerParams(use_tc_tiling_on_sc=True),
        scratch_types=[
            pltpu.VMEM((_BPW,), jnp.int32),
            pltpu.VMEM((_BPW, 128), jnp.float32),
            pltpu.SemaphoreType.DMA,
        ],
    )(n4, pa, pb)


def _mlp_body(gug, gig, gum, gim, ku, ki,
              W0u4, W0i4, b0, W1, b1, W2, b2, W3, b3, Wpg4, Wph, bp, out):
    lane_blk = lax.broadcasted_iota(jnp.int32, (1, 128), 1) // EMB_DIM
    mu_v = (ku[...] == lane_blk).astype(jnp.float32)
    mi_v = (ki[...] == lane_blk).astype(jnp.float32)
    mm = lambda a, b: lax.dot_general(
        a, b, (((1,), (0,)), ((), ())), preferred_element_type=jnp.float32)
    h = jnp.maximum(
        mm(gum[...] * mu_v, W0u4[...]) + mm(gim[...] * mi_v, W0i4[...])
        + b0[...], 0.0)
    for W, b in ((W1, b1), (W2, b2), (W3, b3)):
        h = jnp.maximum(
            lax.dot_general(h, W[...], (((1,), (1,)), ((), ())),
                            preferred_element_type=jnp.float32) + b[...],
            0.0)
    mig = gig[...] * mi_v
    ig_rep = mm(mig, jnp.tile(jnp.eye(EMB_DIM, dtype=jnp.float32), (4, 4)))
    s = (gug[...] * mu_v) * ig_rep
    logits = (mm(s, Wpg4[...]) +
              lax.dot_general(h, Wph[...], (((1,), (1,)), ((), ())),
                              preferred_element_type=jnp.float32)
              + bp[...].reshape(1, 1))
    out[...] = jax.nn.sigmoid(logits)


def _tc_mlp(gug, gig, gum, gim, ku, ki,
            W0u4, W0i4, b0, W1, b1, W2, b2, W3, b3, Wpg4, Wph, bp,
            interpret=False):
    act = pl.BlockSpec((_BB, 128), lambda b: (b, 0))
    col = pl.BlockSpec((_BB, 1), lambda b: (b, 0))
    full2 = lambda a: pl.BlockSpec(a.shape, lambda b: (0,) * a.ndim)
    grid = BATCH_N // _BB
    return pl.pallas_call(
        _mlp_body,
        grid=(grid,),
        in_specs=[act, act, act, act, col, col,
                  full2(W0u4), full2(W0i4), full2(b0), full2(W1), full2(b1),
                  full2(W2), full2(b2), full2(W3), full2(b3),
                  full2(Wpg4), full2(Wph), full2(bp)],
        out_specs=pl.BlockSpec((_BB, 1), lambda b: (b, 0)),
        out_shape=jax.ShapeDtypeStruct((BATCH_N, 1), jnp.float32),
        interpret=interpret,
    )(gug, gig, gum, gim, ku, ki,
      W0u4, W0i4, b0, W1, b1, W2, b2, W3, b3, Wpg4, Wph, bp)


def kernel(u, i, user_gmf, item_gmf, user_mlp, item_mlp,
           W0, b0, W1, b1, W2, b2, W3, b3, Wp, bp):
    u = u.astype(jnp.int32)
    i = i.astype(jnp.int32)
    u4 = (u // _JB) * _SB + (u % _SB)
    i4 = (i // _JB) * _SB + (i % _SB)
    ku = ((u % _JB) // _SB).reshape(BATCH_N, 1)
    ki = ((i % _JB) // _SB).reshape(BATCH_N, 1)
    W0u4 = jnp.tile(W0[:, :EMB_DIM].T, (4, 1))
    W0i4 = jnp.tile(W0[:, EMB_DIM:].T, (4, 1))
    Wpg4 = jnp.tile(Wp[:, :EMB_DIM].T, (4, 1))
    Wph = Wp[:, EMB_DIM:]
    pug, pum = _repack2(user_gmf.T, user_mlp.T)
    gug, gum = _sc_gather2(u4, pug, pum)
    pig, pim = _repack2(item_gmf.T, item_mlp.T)
    gig, gim = _sc_gather2(i4, pig, pim)
    out = _tc_mlp(gug, gig, gum, gim, ku, ki,
                  W0u4, W0i4, b0, W1, b1, W2, b2, W3, b3, Wpg4, Wph, bp)
    return jnp.squeeze(out, axis=-1)
